# baseline (speedup 1.0000x reference)
import functools
import math

import jax
import jax.numpy as jnp
from jax import lax
from jax.experimental import pallas as pl
from jax.experimental.pallas import tpu as pltpu
from jax.experimental.pallas import tpu_sc as plsc

F = 128
RBF = 20
CUT = 5.0
NBR = 16
NC, NS = 2, 16
NW = NC * NS
NP = 10240
E = NP * NBR
NPW = NP // NW
EPW = E // NW

@functools.cache
def _mesh():
    return plsc.VectorSubcoreMesh(
        core_axis_name="c", subcore_axis_name="s", num_cores=NC, num_subcores=NS)

_HI = jax.lax.Precision.DEFAULT


def _f32(*shape):
    return jax.ShapeDtypeStruct(shape, jnp.float32)


_PG_EC = 256


def _posg_body(nbr_hbm, px_hbm, py_hbm, pz_hbm, ox_hbm, oy_hbm, oz_hbm,
               idx_v0, idx_v1, bx0, by0, bz0, bx1, by1, bz1,
               sem0, sem1, osem0, osem1):
    wid = lax.axis_index("s") * NC + lax.axis_index("c")
    idxs = (idx_v0, idx_v1)
    bufs = ((bx0, by0, bz0), (bx1, by1, bz1))
    srcs = (px_hbm, py_hbm, pz_hbm)
    outs = (ox_hbm, oy_hbm, oz_hbm)
    sems = (sem0, sem1)
    osems = (osem0, osem1)
    npair = EPW // _PG_EC // 2

    def issue(t, s):
        eb = wid * EPW + t * _PG_EC

        @pl.when(t > 1)
        def _():
            for c in range(3):
                pltpu.make_async_copy(ox_hbm.at[pl.ds(0, _PG_EC)],
                                      bufs[s][c], osems[s]).wait()
        pltpu.sync_copy(nbr_hbm.at[pl.ds(eb, _PG_EC)], idxs[s])
        for c in range(3):
            pltpu.async_copy(srcs[c].at[idxs[s]], bufs[s][c], sems[s])

    def flush(t, s):
        eb = wid * EPW + t * _PG_EC
        for c in range(3):
            pltpu.make_async_copy(srcs[c].at[idxs[s]], bufs[s][c],
                                  sems[s]).wait()
        for c in range(3):
            pltpu.async_copy(bufs[s][c], outs[c].at[pl.ds(eb, _PG_EC)],
                             osems[s])

    issue(0, 0)
    issue(1, 1)

    def pair(i, carry):
        t0 = 2 * i
        flush(t0, 0)

        @pl.when(i < npair - 1)
        def _():
            issue(t0 + 2, 0)
        flush(t0 + 1, 1)

        @pl.when(i < npair - 1)
        def _():
            issue(t0 + 3, 1)
        return carry

    lax.fori_loop(0, npair, pair, 0)
    for s in range(2):
        for c in range(3):
            pltpu.make_async_copy(ox_hbm.at[pl.ds(0, _PG_EC)],
                                  bufs[s][c], osems[s]).wait()


@functools.cache
def _get_posg():
    buf = pltpu.VMEM((_PG_EC,), jnp.float32)
    return functools.partial(
        pl.kernel,
        out_type=[_f32(E), _f32(E), _f32(E)],
        mesh=_mesh(),
        scratch_types=[
            pltpu.VMEM((_PG_EC,), jnp.int32),
            pltpu.VMEM((_PG_EC,), jnp.int32),
            buf, buf, buf, buf, buf, buf,
            pltpu.SemaphoreType.DMA,
            pltpu.SemaphoreType.DMA,
            pltpu.SemaphoreType.DMA,
            pltpu.SemaphoreType.DMA,
        ],
    )(_posg_body)


def _posg(nbr_flat, px, py, pz):
    return _get_posg()(nbr_flat, px, py, pz)


_C = 1
_EC = _C * NBR
_NS = 5


def _msg_body(with_v, nbr_hbm, filt_hbm, dir_hbm, h_hbm, v0, v1, v2,
              ds_hbm, dv0, dv1, dv2,
              idx_all, f_b, h_b, v_b, d_b, ds_ob, dv_ob,
              sem0, sem1, sem2, sem3, sem4,
              osem0, osem1, osem2, osem3, osem4):
    wid = lax.axis_index("s") * NC + lax.axis_index("c")
    sems = (sem0, sem1, sem2, sem3, sem4)
    osems = (osem0, osem1, osem2, osem3, osem4)
    nchunk = NPW // _C
    ngroup = nchunk // _NS
    dvs = (dv0, dv1, dv2)

    pltpu.sync_copy(nbr_hbm.at[pl.ds(wid * EPW, EPW)], idx_all)

    def issue(t, s):
        eb = wid * EPW + t * _EC
        idx = idx_all.at[pl.ds(t * _EC, _EC)]
        pltpu.async_copy(dir_hbm.at[pl.ds(eb, _EC)], d_b.at[s], sems[s])
        pltpu.async_copy(filt_hbm.at[pl.ds(eb, _EC)], f_b.at[s], sems[s])
        pltpu.async_copy(h_hbm.at[idx], h_b.at[s], sems[s])
        if with_v:
            pltpu.async_copy(v0.at[idx], v_b.at[s, 0], sems[s])
            pltpu.async_copy(v1.at[idx], v_b.at[s, 1], sems[s])
            pltpu.async_copy(v2.at[idx], v_b.at[s, 2], sems[s])

    def compute(t, s):
        eb = wid * EPW + t * _EC
        nb = wid * NPW + t * _C
        pltpu.make_async_copy(dir_hbm.at[pl.ds(eb, _EC)], d_b.at[s],
                              sems[s]).wait()
        pltpu.make_async_copy(filt_hbm.at[pl.ds(eb, _EC)], f_b.at[s],
                              sems[s]).wait()
        pltpu.make_async_copy(h_hbm.at[pl.ds(eb, _EC)], h_b.at[s],
                              sems[s]).wait()
        if with_v:
            for d in range(3):
                pltpu.make_async_copy(v0.at[pl.ds(eb, _EC)], v_b.at[s, d],
                                      sems[s]).wait()

        accs = []
        for n in range(_C):
            def kbody(k, acc):
                e = n * NBR + k
                drow = d_b[s, e, pl.ds(0, 16)]
                dx = drow[0]
                dy = drow[1]
                dz = drow[2]
                a = list(acc)
                for j in range(8):
                    fv = f_b[s, e, pl.ds(j * 16, 16)]
                    hv = h_b[s, e, pl.ds(j * 16, 16)]
                    a[j] = a[j] + fv * hv
                for j in range(8):
                    fv = f_b[s, e, pl.ds(128 + j * 16, 16)]
                    hv = h_b[s, e, pl.ds(128 + j * 16, 16)]
                    tv = fv * hv
                    a[8 + j] = a[8 + j] + tv * dx
                    a[16 + j] = a[16 + j] + tv * dy
                    a[24 + j] = a[24 + j] + tv * dz
                if with_v:
                    for j in range(8):
                        fv = f_b[s, e, pl.ds(256 + j * 16, 16)]
                        hv = h_b[s, e, pl.ds(256 + j * 16, 16)]
                        tv = fv * hv
                        a[8 + j] = a[8 + j] + tv * v_b[s, 0, e, pl.ds(j * 16, 16)]
                        a[16 + j] = a[16 + j] + tv * v_b[s, 1, e, pl.ds(j * 16, 16)]
                        a[24 + j] = a[24 + j] + tv * v_b[s, 2, e, pl.ds(j * 16, 16)]
                return tuple(a)

            zero = jnp.zeros((16,), jnp.float32)
            accs.append(lax.fori_loop(0, NBR, kbody, (zero,) * 32))

        @pl.when(t >= _NS)
        def _():
            pltpu.make_async_copy(ds_hbm.at[pl.ds(nb, _C)], ds_ob.at[s],
                                  osems[s]).wait()
            for d in range(3):
                pltpu.make_async_copy(dv0.at[pl.ds(nb, _C)], dv_ob.at[s, d],
                                      osems[s]).wait()
        for n in range(_C):
            acc = accs[n]
            for j in range(8):
                ds_ob[s, n, pl.ds(j * 16, 16)] = acc[j]
            for d in range(3):
                for j in range(8):
                    dv_ob[s, d, n, pl.ds(j * 16, 16)] = acc[8 + 8 * d + j]
        pltpu.async_copy(ds_ob.at[s], ds_hbm.at[pl.ds(nb, _C)], osems[s])
        for d in range(3):
            pltpu.async_copy(dv_ob.at[s, d], dvs[d].at[pl.ds(nb, _C)], osems[s])

    for s in range(_NS):
        issue(s, s)

    def group(g, carry):
        t0 = _NS * g
        for s in range(_NS):
            compute(t0 + s, s)

            @pl.when(t0 + s + _NS < nchunk)
            def _():
                issue(t0 + s + _NS, s)
        return carry

    lax.fori_loop(0, ngroup, group, 0)
    for s in range(_NS):
        pltpu.make_async_copy(ds_hbm.at[pl.ds(wid * NPW, _C)], ds_ob.at[s],
                              osems[s]).wait()
        for d in range(3):
            pltpu.make_async_copy(dv0.at[pl.ds(wid * NPW, _C)], dv_ob.at[s, d],
                                  osems[s]).wait()


@functools.cache
def _make_msg(with_v):
    return functools.partial(
        pl.kernel,
        out_type=[_f32(NP, F), _f32(NP, F), _f32(NP, F), _f32(NP, F)],
        mesh=_mesh(),
        scratch_types=[
            pltpu.VMEM((EPW,), jnp.int32),
            pltpu.VMEM((_NS, _EC, 3 * F), jnp.float32),
            pltpu.VMEM((_NS, _EC, 3 * F), jnp.float32),
            pltpu.VMEM((_NS, 3, _EC, F), jnp.float32),
            pltpu.VMEM((_NS, _EC, 16), jnp.float32),
            pltpu.VMEM((_NS, _C, F), jnp.float32),
            pltpu.VMEM((_NS, 3, _C, F), jnp.float32),
        ] + [pltpu.SemaphoreType.DMA] * (2 * _NS),
    )(functools.partial(_msg_body, with_v))


def _msg_v(*args):
    return _make_msg(True)(*args)


def _msg_nov(*args):
    return _make_msg(False)(*args)


_BG = 256


def _geo_body(jx_ref, jy_ref, jz_ref, ix_ref, iy_ref, iz_ref, mask_ref,
              nf_ref, embW_ref, embb_ref,
              phiT_ref, dx_ref, dy_ref, dz_ref, s0_ref):
    dvx = jx_ref[...] - ix_ref[...]
    dvy = jy_ref[...] - iy_ref[...]
    dvz = jz_ref[...] - iz_ref[...]
    mask = mask_ref[...]
    d2 = dvx * dvx + dvy * dvy + dvz * dvz
    rij = jnp.sqrt(d2 + 1e-12)
    rij = jnp.where(mask > 0, rij, 0.0)
    safe = jnp.where(rij > 0, rij, 1.0)
    dx_ref[...] = dvx / safe
    dy_ref[...] = dvy / safe
    dz_ref[...] = dvz / safe
    theta = rij * (math.pi / CUT)
    s1 = jnp.sin(theta)
    c1 = jnp.cos(theta)
    fcut = 0.5 * (c1 + 1.0) * (rij < CUT).astype(jnp.float32) * mask
    w = fcut / jnp.where(rij == 0, 1.0, rij)
    two_c = 2.0 * c1
    sprev = jnp.zeros_like(s1)
    scur = s1
    for r in range(RBF):
        phiT_ref[r] = scur * w
        snext = two_c * scur - sprev
        sprev = scur
        scur = snext
    phiT_ref[RBF] = fcut
    zero = jnp.zeros_like(s1)
    phiT_ref[RBF + 1] = zero
    phiT_ref[RBF + 2] = zero
    phiT_ref[RBF + 3] = zero
    s0_ref[...] = jnp.dot(nf_ref[...], embW_ref[...],
                          preferred_element_type=jnp.float32,
                          precision=_HI) + embb_ref[...]


_EB = _BG * NBR // 128


def _geo_emb(jx, jy, jz, ix, iy, iz, mask, nf, embW, embb):
    g = NP // _BG
    epk = pl.BlockSpec((_EB, 128), lambda i: (i, 0))
    return pl.pallas_call(
        _geo_body,
        grid=(g,),
        in_specs=[
            epk, epk, epk, epk, epk, epk, epk,
            pl.BlockSpec((_BG, 128), lambda i: (i, 0)),
            pl.BlockSpec((128, F), lambda i: (0, 0)),
            pl.BlockSpec((1, F), lambda i: (0, 0)),
        ],
        out_specs=[
            pl.BlockSpec((24, _EB, 128), lambda i: (0, i, 0)),
            epk, epk, epk,
            pl.BlockSpec((_BG, F), lambda i: (i, 0)),
        ],
        out_shape=[
            _f32(24, E // 128, 128), _f32(E // 128, 128),
            _f32(E // 128, 128), _f32(E // 128, 128), _f32(NP, F),
        ],
    )(jx, jy, jz, ix, iy, iz, mask, nf, embW, embb)


_BP = 256


def _pre_body(s_ref, phi_ref, W1_ref, b1_ref, W2_ref, b2_ref, Waug_ref,
              h_ref, filt_ref):
    s = s_ref[...]
    h = jnp.dot(s, W1_ref[...], preferred_element_type=jnp.float32,
                precision=_HI) + b1_ref[...]
    h = h * jax.nn.sigmoid(h)
    h_ref[...] = jnp.dot(h, W2_ref[...], preferred_element_type=jnp.float32,
                         precision=_HI) + b2_ref[...]
    filt_ref[...] = jax.lax.dot_general(
        phi_ref[...], Waug_ref[...], (((0,), (0,)), ((), ())),
        preferred_element_type=jnp.float32, precision=_HI)


def _pre(scalars, phi_aug, W1, b1, W2, b2, Waug):
    g = NP // _BP
    return pl.pallas_call(
        _pre_body,
        grid=(g,),
        in_specs=[
            pl.BlockSpec((_BP, F), lambda i: (i, 0)),
            pl.BlockSpec((24, _BP * NBR), lambda i: (0, i)),
            pl.BlockSpec((F, F), lambda i: (0, 0)),
            pl.BlockSpec((1, F), lambda i: (0, 0)),
            pl.BlockSpec((F, 3 * F), lambda i: (0, 0)),
            pl.BlockSpec((1, 3 * F), lambda i: (0, 0)),
            pl.BlockSpec((24, 3 * F), lambda i: (0, 0)),
        ],
        out_specs=[
            pl.BlockSpec((_BP, 3 * F), lambda i: (i, 0)),
            pl.BlockSpec((_BP * NBR, 3 * F), lambda i: (i, 0)),
        ],
        out_shape=[_f32(NP, 3 * F), _f32(E, 3 * F)],
    )(scalars, phi_aug, W1, b1, W2, b2, Waug)


def _post_body(s_ref, v_ref, ds_ref, dv_ref, muW_ref, W1_ref, b1_ref,
               W2_ref, b2_ref, so_ref, vo_ref):
    s = s_ref[...] + ds_ref[...]
    muW = muW_ref[...]
    v = [v_ref[d] + dv_ref[d] for d in range(3)]
    mix = [jnp.dot(v[d], muW, preferred_element_type=jnp.float32,
                   precision=_HI) for d in range(3)]
    V = [m[:, :F] for m in mix]
    U = [m[:, F:] for m in mix]
    mu_Vn = jnp.sqrt(V[0] * V[0] + V[1] * V[1] + V[2] * V[2] + 1e-12)
    ctx = jnp.concatenate([s, mu_Vn], axis=-1)
    h2 = jnp.dot(ctx, W1_ref[...], preferred_element_type=jnp.float32,
                 precision=_HI) + b1_ref[...]
    h2 = h2 * jax.nn.sigmoid(h2)
    h2b = jnp.dot(h2, W2_ref[...], preferred_element_type=jnp.float32,
                  precision=_HI) + b2_ref[...]
    ds2 = h2b[:, :F]
    dv2 = h2b[:, F:2 * F]
    dsv = h2b[:, 2 * F:]
    VU = V[0] * U[0] + V[1] * U[1] + V[2] * U[2]
    so_ref[...] = s + ds2 + dsv * VU
    for d in range(3):
        vo_ref[d] = v[d] + dv2 * U[d]


def _post(s, v, ds, dv, muW, W1, b1, W2, b2):
    g = NP // _BP
    return pl.pallas_call(
        _post_body,
        grid=(g,),
        in_specs=[
            pl.BlockSpec((_BP, F), lambda i: (i, 0)),
            pl.BlockSpec((3, _BP, F), lambda i: (0, i, 0)),
            pl.BlockSpec((_BP, F), lambda i: (i, 0)),
            pl.BlockSpec((3, _BP, F), lambda i: (0, i, 0)),
            pl.BlockSpec((F, 2 * F), lambda i: (0, 0)),
            pl.BlockSpec((2 * F, F), lambda i: (0, 0)),
            pl.BlockSpec((1, F), lambda i: (0, 0)),
            pl.BlockSpec((F, 3 * F), lambda i: (0, 0)),
            pl.BlockSpec((1, 3 * F), lambda i: (0, 0)),
        ],
        out_specs=[
            pl.BlockSpec((_BP, F), lambda i: (i, 0)),
            pl.BlockSpec((3, _BP, F), lambda i: (0, i, 0)),
        ],
        out_shape=[_f32(NP, F), _f32(3, NP, F)],
    )(s, v, ds, dv, muW, W1, b1, W2, b2)


def _postpre_body(s_ref, v_ref, ds_ref, dv_ref, muW_ref, W1_ref, b1_ref,
                  W2_ref, b2_ref, phi_ref, nW1_ref, nb1_ref, nW2_ref,
                  nb2_ref, nWaug_ref, so_ref, vo_ref, h_ref, filt_ref):
    s = s_ref[...] + ds_ref[...]
    muW = muW_ref[...]
    v = [v_ref[d] + dv_ref[d] for d in range(3)]
    mix = [jnp.dot(v[d], muW, preferred_element_type=jnp.float32,
                   precision=_HI) for d in range(3)]
    V = [m[:, :F] for m in mix]
    U = [m[:, F:] for m in mix]
    mu_Vn = jnp.sqrt(V[0] * V[0] + V[1] * V[1] + V[2] * V[2] + 1e-12)
    ctx = jnp.concatenate([s, mu_Vn], axis=-1)
    h2 = jnp.dot(ctx, W1_ref[...], preferred_element_type=jnp.float32,
                 precision=_HI) + b1_ref[...]
    h2 = h2 * jax.nn.sigmoid(h2)
    h2b = jnp.dot(h2, W2_ref[...], preferred_element_type=jnp.float32,
                  precision=_HI) + b2_ref[...]
    ds2 = h2b[:, :F]
    dv2 = h2b[:, F:2 * F]
    dsv = h2b[:, 2 * F:]
    VU = V[0] * U[0] + V[1] * U[1] + V[2] * U[2]
    snew = s + ds2 + dsv * VU
    so_ref[...] = snew
    for d in range(3):
        vo_ref[d] = v[d] + dv2 * U[d]
    hn = jnp.dot(snew, nW1_ref[...], preferred_element_type=jnp.float32,
                 precision=_HI) + nb1_ref[...]
    hn = hn * jax.nn.sigmoid(hn)
    h_ref[...] = jnp.dot(hn, nW2_ref[...], preferred_element_type=jnp.float32,
                         precision=_HI) + nb2_ref[...]
    filt_ref[...] = jax.lax.dot_general(
        phi_ref[...], nWaug_ref[...], (((0,), (0,)), ((), ())),
        preferred_element_type=jnp.float32, precision=_HI)


def _postpre(s, v, ds, dv, muW, W1, b1, W2, b2, phi, nW1, nb1, nW2, nb2, nWaug):
    g = NP // _BP
    return pl.pallas_call(
        _postpre_body,
        grid=(g,),
        in_specs=[
            pl.BlockSpec((_BP, F), lambda i: (i, 0)),
            pl.BlockSpec((3, _BP, F), lambda i: (0, i, 0)),
            pl.BlockSpec((_BP, F), lambda i: (i, 0)),
            pl.BlockSpec((3, _BP, F), lambda i: (0, i, 0)),
            pl.BlockSpec((F, 2 * F), lambda i: (0, 0)),
            pl.BlockSpec((2 * F, F), lambda i: (0, 0)),
            pl.BlockSpec((1, F), lambda i: (0, 0)),
            pl.BlockSpec((F, 3 * F), lambda i: (0, 0)),
            pl.BlockSpec((1, 3 * F), lambda i: (0, 0)),
            pl.BlockSpec((24, _BP * NBR), lambda i: (0, i)),
            pl.BlockSpec((F, F), lambda i: (0, 0)),
            pl.BlockSpec((1, F), lambda i: (0, 0)),
            pl.BlockSpec((F, 3 * F), lambda i: (0, 0)),
            pl.BlockSpec((1, 3 * F), lambda i: (0, 0)),
            pl.BlockSpec((24, 3 * F), lambda i: (0, 0)),
        ],
        out_specs=[
            pl.BlockSpec((_BP, F), lambda i: (i, 0)),
            pl.BlockSpec((3, _BP, F), lambda i: (0, i, 0)),
            pl.BlockSpec((_BP, 3 * F), lambda i: (i, 0)),
            pl.BlockSpec((_BP * NBR, 3 * F), lambda i: (i, 0)),
        ],
        out_shape=[_f32(NP, F), _f32(3, NP, F),
                   _f32(NP, 3 * F), _f32(E, 3 * F)],
    )(s, v, ds, dv, muW, W1, b1, W2, b2, phi, nW1, nb1, nW2, nb2, nWaug)


def kernel(node_features, positions, neighbors, neighbor_mask, atom_mask, params):
    nf = node_features[0]
    pos = positions[0]
    nbr = neighbors[0].astype(jnp.int32)
    mask = neighbor_mask[0]
    N = nf.shape[0]
    pad = NP - N

    nf = jnp.pad(nf, ((0, pad), (0, 28)))
    nbr = jnp.pad(nbr, ((0, pad), (0, 0)))
    mask = jnp.pad(mask, ((0, pad), (0, 0)))
    nbr_flat = nbr.reshape(E)

    jx, jy, jz = _posg(nbr_flat,
                       jnp.pad(pos[:, 0], (0, pad)),
                       jnp.pad(pos[:, 1], (0, pad)),
                       jnp.pad(pos[:, 2], (0, pad)))
    EP = E // 128

    def _rep(col):
        return jnp.broadcast_to(col[:, None], (NP, NBR)).reshape(EP, 128)

    phiT, dx, dy, dz, scalars = _geo_emb(
        jx.reshape(EP, 128), jy.reshape(EP, 128), jz.reshape(EP, 128),
        _rep(jnp.pad(pos[:, 0], (0, pad))),
        _rep(jnp.pad(pos[:, 1], (0, pad))),
        _rep(jnp.pad(pos[:, 2], (0, pad))),
        mask.reshape(EP, 128), nf,
        jnp.pad(params['emb_W'], ((0, 28), (0, 0))),
        params['emb_b'][None])

    phiT2 = phiT.reshape(24, E)
    dir16 = jnp.pad(jnp.stack(
        [dx.reshape(E), dy.reshape(E), dz.reshape(E)], axis=-1),
        ((0, 0), (0, 13)))

    vec = jnp.zeros((3, NP, F), jnp.float32)
    zeros_pf = jnp.zeros((NP, F), jnp.float32)

    def waug(l):
        W_l = params['filter_W'][:, l * 3 * F:(l + 1) * 3 * F]
        b_l = params['filter_b'][l * 3 * F:(l + 1) * 3 * F]
        return jnp.concatenate(
            [W_l, b_l[None], jnp.zeros((3, 3 * F), jnp.float32)], axis=0)

    h_i, filt = _pre(scalars, phiT2,
                     params['inter0_W1'], params['inter0_b1'][None],
                     params['inter0_W2'], params['inter0_b2'][None],
                     waug(0))
    for l in range(3):
        if l == 0:
            ds, d0, d1, d2 = _msg_nov(nbr_flat, filt, dir16, h_i,
                                      zeros_pf, zeros_pf, zeros_pf)
        else:
            ds, d0, d1, d2 = _msg_v(nbr_flat, filt, dir16, h_i,
                                    vec[0], vec[1], vec[2])
        dv = jnp.stack([d0, d1, d2])
        if l < 2:
            nl = l + 1
            scalars, vec, h_i, filt = _postpre(
                scalars, vec, ds, dv,
                params['mu%d_W' % l],
                params['intra%d_W1' % l], params['intra%d_b1' % l][None],
                params['intra%d_W2' % l], params['intra%d_b2' % l][None],
                phiT2,
                params['inter%d_W1' % nl], params['inter%d_b1' % nl][None],
                params['inter%d_W2' % nl], params['inter%d_b2' % nl][None],
                waug(nl))
        else:
            scalars, vec = _post(scalars, vec, ds, dv,
                                 params['mu%d_W' % l],
                                 params['intra%d_W1' % l],
                                 params['intra%d_b1' % l][None],
                                 params['intra%d_W2' % l],
                                 params['intra%d_b2' % l][None])
    return scalars[:N][None]


# speedup vs baseline: 1.0310x; 1.0310x over previous
import functools
import math

import jax
import jax.numpy as jnp
from jax import lax
from jax.experimental import pallas as pl
from jax.experimental.pallas import tpu as pltpu
from jax.experimental.pallas import tpu_sc as plsc

F = 128
RBF = 20
CUT = 5.0
NBR = 16
NC, NS = 2, 16
NW = NC * NS
NP = 10240
E = NP * NBR
NPW = NP // NW
EPW = E // NW

@functools.cache
def _mesh():
    return plsc.VectorSubcoreMesh(
        core_axis_name="c", subcore_axis_name="s", num_cores=NC, num_subcores=NS)

_HI = jax.lax.Precision.DEFAULT


def _f32(*shape):
    return jax.ShapeDtypeStruct(shape, jnp.float32)


_PG_EC = 256


def _posg_body(nbr_hbm, px_hbm, py_hbm, pz_hbm, ox_hbm, oy_hbm, oz_hbm,
               idx_v0, idx_v1, bx0, by0, bz0, bx1, by1, bz1,
               sem0, sem1, osem0, osem1):
    wid = lax.axis_index("s") * NC + lax.axis_index("c")
    idxs = (idx_v0, idx_v1)
    bufs = ((bx0, by0, bz0), (bx1, by1, bz1))
    srcs = (px_hbm, py_hbm, pz_hbm)
    outs = (ox_hbm, oy_hbm, oz_hbm)
    sems = (sem0, sem1)
    osems = (osem0, osem1)
    npair = EPW // _PG_EC // 2

    def issue(t, s):
        eb = wid * EPW + t * _PG_EC

        @pl.when(t > 1)
        def _():
            for c in range(3):
                pltpu.make_async_copy(ox_hbm.at[pl.ds(0, _PG_EC)],
                                      bufs[s][c], osems[s]).wait()
        pltpu.sync_copy(nbr_hbm.at[pl.ds(eb, _PG_EC)], idxs[s])
        for c in range(3):
            pltpu.async_copy(srcs[c].at[idxs[s]], bufs[s][c], sems[s])

    def flush(t, s):
        eb = wid * EPW + t * _PG_EC
        for c in range(3):
            pltpu.make_async_copy(srcs[c].at[idxs[s]], bufs[s][c],
                                  sems[s]).wait()
        for c in range(3):
            pltpu.async_copy(bufs[s][c], outs[c].at[pl.ds(eb, _PG_EC)],
                             osems[s])

    issue(0, 0)
    issue(1, 1)

    def pair(i, carry):
        t0 = 2 * i
        flush(t0, 0)

        @pl.when(i < npair - 1)
        def _():
            issue(t0 + 2, 0)
        flush(t0 + 1, 1)

        @pl.when(i < npair - 1)
        def _():
            issue(t0 + 3, 1)
        return carry

    lax.fori_loop(0, npair, pair, 0)
    for s in range(2):
        for c in range(3):
            pltpu.make_async_copy(ox_hbm.at[pl.ds(0, _PG_EC)],
                                  bufs[s][c], osems[s]).wait()


@functools.cache
def _get_posg():
    buf = pltpu.VMEM((_PG_EC,), jnp.float32)
    return functools.partial(
        pl.kernel,
        out_type=[_f32(E), _f32(E), _f32(E)],
        mesh=_mesh(),
        scratch_types=[
            pltpu.VMEM((_PG_EC,), jnp.int32),
            pltpu.VMEM((_PG_EC,), jnp.int32),
            buf, buf, buf, buf, buf, buf,
            pltpu.SemaphoreType.DMA,
            pltpu.SemaphoreType.DMA,
            pltpu.SemaphoreType.DMA,
            pltpu.SemaphoreType.DMA,
        ],
    )(_posg_body)


def _posg(nbr_flat, px, py, pz):
    return _get_posg()(nbr_flat, px, py, pz)


_C = 1
_EC = _C * NBR
_NS = 5


def _msg_body(with_v, nbr_hbm, filt_hbm, dir_hbm, h_hbm, v0, v1, v2,
              ds_hbm, dv0, dv1, dv2,
              idx_all, f_b, h_b, v_b, d_b, ds_ob, dv_ob,
              sem0, sem1, sem2, sem3, sem4,
              osem0, osem1, osem2, osem3, osem4):
    wid = lax.axis_index("s") * NC + lax.axis_index("c")
    sems = (sem0, sem1, sem2, sem3, sem4)
    osems = (osem0, osem1, osem2, osem3, osem4)
    nchunk = NPW // _C
    ngroup = nchunk // _NS
    dvs = (dv0, dv1, dv2)

    pltpu.sync_copy(nbr_hbm.at[pl.ds(wid * EPW, EPW)], idx_all)

    def issue(t, s):
        eb = wid * EPW + t * _EC
        idx = idx_all.at[pl.ds(t * _EC, _EC)]
        pltpu.async_copy(dir_hbm.at[pl.ds(eb, _EC)], d_b.at[s], sems[s])
        pltpu.async_copy(filt_hbm.at[pl.ds(eb, _EC)], f_b.at[s], sems[s])
        pltpu.async_copy(h_hbm.at[idx], h_b.at[s], sems[s])
        if with_v:
            pltpu.async_copy(v0.at[idx], v_b.at[s, 0], sems[s])
            pltpu.async_copy(v1.at[idx], v_b.at[s, 1], sems[s])
            pltpu.async_copy(v2.at[idx], v_b.at[s, 2], sems[s])

    def compute(t, s):
        eb = wid * EPW + t * _EC
        nb = wid * NPW + t * _C
        pltpu.make_async_copy(dir_hbm.at[pl.ds(eb, _EC)], d_b.at[s],
                              sems[s]).wait()
        pltpu.make_async_copy(filt_hbm.at[pl.ds(eb, _EC)], f_b.at[s],
                              sems[s]).wait()
        pltpu.make_async_copy(h_hbm.at[pl.ds(eb, _EC)], h_b.at[s],
                              sems[s]).wait()
        if with_v:
            for d in range(3):
                pltpu.make_async_copy(v0.at[pl.ds(eb, _EC)], v_b.at[s, d],
                                      sems[s]).wait()

        accs = []
        for n in range(_C):
            def kbody(k, acc):
                e = n * NBR + k
                drow = d_b[s, e, pl.ds(0, 16)]
                dx = drow[0]
                dy = drow[1]
                dz = drow[2]
                a = list(acc)
                for j in range(8):
                    fv = f_b[s, e, pl.ds(j * 16, 16)]
                    hv = h_b[s, e, pl.ds(j * 16, 16)]
                    a[j] = a[j] + fv * hv
                for j in range(8):
                    fv = f_b[s, e, pl.ds(128 + j * 16, 16)]
                    hv = h_b[s, e, pl.ds(128 + j * 16, 16)]
                    tv = fv * hv
                    a[8 + j] = a[8 + j] + tv * dx
                    a[16 + j] = a[16 + j] + tv * dy
                    a[24 + j] = a[24 + j] + tv * dz
                if with_v:
                    for j in range(8):
                        fv = f_b[s, e, pl.ds(256 + j * 16, 16)]
                        hv = h_b[s, e, pl.ds(256 + j * 16, 16)]
                        tv = fv * hv
                        a[8 + j] = a[8 + j] + tv * v_b[s, 0, e, pl.ds(j * 16, 16)]
                        a[16 + j] = a[16 + j] + tv * v_b[s, 1, e, pl.ds(j * 16, 16)]
                        a[24 + j] = a[24 + j] + tv * v_b[s, 2, e, pl.ds(j * 16, 16)]
                return tuple(a)

            zero = jnp.zeros((16,), jnp.float32)
            accs.append(lax.fori_loop(0, NBR, kbody, (zero,) * 32))

        @pl.when(t >= _NS)
        def _():
            pltpu.make_async_copy(ds_hbm.at[pl.ds(nb, _C)], ds_ob.at[s],
                                  osems[s]).wait()
            for d in range(3):
                pltpu.make_async_copy(dv0.at[pl.ds(nb, _C)], dv_ob.at[s, d],
                                      osems[s]).wait()
        for n in range(_C):
            acc = accs[n]
            for j in range(8):
                ds_ob[s, n, pl.ds(j * 16, 16)] = acc[j]
            for d in range(3):
                for j in range(8):
                    dv_ob[s, d, n, pl.ds(j * 16, 16)] = acc[8 + 8 * d + j]
        pltpu.async_copy(ds_ob.at[s], ds_hbm.at[pl.ds(nb, _C)], osems[s])
        for d in range(3):
            pltpu.async_copy(dv_ob.at[s, d], dvs[d].at[pl.ds(nb, _C)], osems[s])

    for s in range(_NS):
        issue(s, s)

    def group(g, carry):
        t0 = _NS * g
        for s in range(_NS):
            compute(t0 + s, s)

            @pl.when(t0 + s + _NS < nchunk)
            def _():
                issue(t0 + s + _NS, s)
        return carry

    lax.fori_loop(0, ngroup, group, 0)
    for s in range(_NS):
        pltpu.make_async_copy(ds_hbm.at[pl.ds(wid * NPW, _C)], ds_ob.at[s],
                              osems[s]).wait()
        for d in range(3):
            pltpu.make_async_copy(dv0.at[pl.ds(wid * NPW, _C)], dv_ob.at[s, d],
                                  osems[s]).wait()


@functools.cache
def _make_msg(with_v):
    return functools.partial(
        pl.kernel,
        out_type=[_f32(NP, F), _f32(NP, F), _f32(NP, F), _f32(NP, F)],
        mesh=_mesh(),
        scratch_types=[
            pltpu.VMEM((EPW,), jnp.int32),
            pltpu.VMEM((_NS, _EC, 3 * F), jnp.float32),
            pltpu.VMEM((_NS, _EC, 3 * F), jnp.float32),
            pltpu.VMEM((_NS, 3, _EC, F), jnp.float32),
            pltpu.VMEM((_NS, _EC, 16), jnp.float32),
            pltpu.VMEM((_NS, _C, F), jnp.float32),
            pltpu.VMEM((_NS, 3, _C, F), jnp.float32),
        ] + [pltpu.SemaphoreType.DMA] * (2 * _NS),
    )(functools.partial(_msg_body, with_v))


def _msg_v(*args):
    return _make_msg(True)(*args)


def _msg_nov(*args):
    return _make_msg(False)(*args)


_BG = 256


def _geo_body(jx_ref, jy_ref, jz_ref, ix_ref, iy_ref, iz_ref, mask_ref,
              nf_ref, embW_ref, embb_ref,
              phiT_ref, dx_ref, dy_ref, dz_ref, s0_ref):
    dvx = jx_ref[...] - ix_ref[...]
    dvy = jy_ref[...] - iy_ref[...]
    dvz = jz_ref[...] - iz_ref[...]
    mask = mask_ref[...]
    d2 = dvx * dvx + dvy * dvy + dvz * dvz
    rij = jnp.sqrt(d2 + 1e-12)
    rij = jnp.where(mask > 0, rij, 0.0)
    safe = jnp.where(rij > 0, rij, 1.0)
    dx_ref[...] = dvx / safe
    dy_ref[...] = dvy / safe
    dz_ref[...] = dvz / safe
    theta = rij * (math.pi / CUT)
    s1 = jnp.sin(theta)
    c1 = jnp.cos(theta)
    fcut = 0.5 * (c1 + 1.0) * (rij < CUT).astype(jnp.float32) * mask
    w = fcut / jnp.where(rij == 0, 1.0, rij)
    two_c = 2.0 * c1
    sprev = jnp.zeros_like(s1)
    scur = s1
    for r in range(RBF):
        phiT_ref[r] = scur * w
        snext = two_c * scur - sprev
        sprev = scur
        scur = snext
    phiT_ref[RBF] = fcut
    zero = jnp.zeros_like(s1)
    phiT_ref[RBF + 1] = zero
    phiT_ref[RBF + 2] = zero
    phiT_ref[RBF + 3] = zero
    s0_ref[...] = jnp.dot(nf_ref[...], embW_ref[...],
                          preferred_element_type=jnp.float32,
                          precision=_HI) + embb_ref[...]


_EB = _BG * NBR // 128


def _geo_emb(jx, jy, jz, ix, iy, iz, mask, nf, embW, embb):
    g = NP // _BG
    epk = pl.BlockSpec((_EB, 128), lambda i: (i, 0))
    return pl.pallas_call(
        _geo_body,
        grid=(g,),
        in_specs=[
            epk, epk, epk, epk, epk, epk, epk,
            pl.BlockSpec((_BG, 128), lambda i: (i, 0)),
            pl.BlockSpec((128, F), lambda i: (0, 0)),
            pl.BlockSpec((1, F), lambda i: (0, 0)),
        ],
        out_specs=[
            pl.BlockSpec((24, _EB, 128), lambda i: (0, i, 0)),
            epk, epk, epk,
            pl.BlockSpec((_BG, F), lambda i: (i, 0)),
        ],
        out_shape=[
            _f32(24, E // 128, 128), _f32(E // 128, 128),
            _f32(E // 128, 128), _f32(E // 128, 128), _f32(NP, F),
        ],
    )(jx, jy, jz, ix, iy, iz, mask, nf, embW, embb)


_BP = 256


def _pre_body(s_ref, phi_ref, W1_ref, b1_ref, W2_ref, b2_ref, Waug_ref,
              h_ref, filt_ref):
    s = s_ref[...]
    h = jnp.dot(s, W1_ref[...], preferred_element_type=jnp.float32,
                precision=_HI) + b1_ref[...]
    h = h * jax.nn.sigmoid(h)
    h_ref[...] = jnp.dot(h, W2_ref[...], preferred_element_type=jnp.float32,
                         precision=_HI) + b2_ref[...]
    filt_ref[...] = jax.lax.dot_general(
        phi_ref[...], Waug_ref[...], (((0,), (0,)), ((), ())),
        preferred_element_type=jnp.float32, precision=_HI)


def _pre(scalars, phi_aug, W1, b1, W2, b2, Waug):
    g = NP // _BP
    return pl.pallas_call(
        _pre_body,
        grid=(g,),
        in_specs=[
            pl.BlockSpec((_BP, F), lambda i: (i, 0)),
            pl.BlockSpec((24, _BP * NBR), lambda i: (0, i)),
            pl.BlockSpec((F, F), lambda i: (0, 0)),
            pl.BlockSpec((1, F), lambda i: (0, 0)),
            pl.BlockSpec((F, 3 * F), lambda i: (0, 0)),
            pl.BlockSpec((1, 3 * F), lambda i: (0, 0)),
            pl.BlockSpec((24, 3 * F), lambda i: (0, 0)),
        ],
        out_specs=[
            pl.BlockSpec((_BP, 3 * F), lambda i: (i, 0)),
            pl.BlockSpec((_BP * NBR, 3 * F), lambda i: (i, 0)),
        ],
        out_shape=[_f32(NP, 3 * F), _f32(E, 3 * F)],
    )(scalars, phi_aug, W1, b1, W2, b2, Waug)


def _post_body(s_ref, v_ref, ds_ref, dv_ref, muW_ref, W1_ref, b1_ref,
               W2_ref, b2_ref, so_ref, vo_ref):
    s = s_ref[...] + ds_ref[...]
    muW = muW_ref[...]
    v = [v_ref[d] + dv_ref[d] for d in range(3)]
    mix = [jnp.dot(v[d], muW, preferred_element_type=jnp.float32,
                   precision=_HI) for d in range(3)]
    V = [m[:, :F] for m in mix]
    U = [m[:, F:] for m in mix]
    mu_Vn = jnp.sqrt(V[0] * V[0] + V[1] * V[1] + V[2] * V[2] + 1e-12)
    ctx = jnp.concatenate([s, mu_Vn], axis=-1)
    h2 = jnp.dot(ctx, W1_ref[...], preferred_element_type=jnp.float32,
                 precision=_HI) + b1_ref[...]
    h2 = h2 * jax.nn.sigmoid(h2)
    h2b = jnp.dot(h2, W2_ref[...], preferred_element_type=jnp.float32,
                  precision=_HI) + b2_ref[...]
    ds2 = h2b[:, :F]
    dv2 = h2b[:, F:2 * F]
    dsv = h2b[:, 2 * F:]
    VU = V[0] * U[0] + V[1] * U[1] + V[2] * U[2]
    so_ref[...] = s + ds2 + dsv * VU
    for d in range(3):
        vo_ref[d] = v[d] + dv2 * U[d]


def _post(s, v, ds, dv, muW, W1, b1, W2, b2):
    g = NP // _BP
    return pl.pallas_call(
        _post_body,
        grid=(g,),
        in_specs=[
            pl.BlockSpec((_BP, F), lambda i: (i, 0)),
            pl.BlockSpec((3, _BP, F), lambda i: (0, i, 0)),
            pl.BlockSpec((_BP, F), lambda i: (i, 0)),
            pl.BlockSpec((3, _BP, F), lambda i: (0, i, 0)),
            pl.BlockSpec((F, 2 * F), lambda i: (0, 0)),
            pl.BlockSpec((2 * F, F), lambda i: (0, 0)),
            pl.BlockSpec((1, F), lambda i: (0, 0)),
            pl.BlockSpec((F, 3 * F), lambda i: (0, 0)),
            pl.BlockSpec((1, 3 * F), lambda i: (0, 0)),
        ],
        out_specs=[
            pl.BlockSpec((_BP, F), lambda i: (i, 0)),
            pl.BlockSpec((3, _BP, F), lambda i: (0, i, 0)),
        ],
        out_shape=[_f32(NP, F), _f32(3, NP, F)],
    )(s, v, ds, dv, muW, W1, b1, W2, b2)


def kernel(node_features, positions, neighbors, neighbor_mask, atom_mask, params):
    nf = node_features[0]
    pos = positions[0]
    nbr = neighbors[0].astype(jnp.int32)
    mask = neighbor_mask[0]
    N = nf.shape[0]
    pad = NP - N

    nf = jnp.pad(nf, ((0, pad), (0, 28)))
    nbr = jnp.pad(nbr, ((0, pad), (0, 0)))
    mask = jnp.pad(mask, ((0, pad), (0, 0)))
    nbr_flat = nbr.reshape(E)

    jx, jy, jz = _posg(nbr_flat,
                       jnp.pad(pos[:, 0], (0, pad)),
                       jnp.pad(pos[:, 1], (0, pad)),
                       jnp.pad(pos[:, 2], (0, pad)))
    EP = E // 128

    def _rep(col):
        return jnp.broadcast_to(col[:, None], (NP, NBR)).reshape(EP, 128)

    phiT, dx, dy, dz, scalars = _geo_emb(
        jx.reshape(EP, 128), jy.reshape(EP, 128), jz.reshape(EP, 128),
        _rep(jnp.pad(pos[:, 0], (0, pad))),
        _rep(jnp.pad(pos[:, 1], (0, pad))),
        _rep(jnp.pad(pos[:, 2], (0, pad))),
        mask.reshape(EP, 128), nf,
        jnp.pad(params['emb_W'], ((0, 28), (0, 0))),
        params['emb_b'][None])

    phiT2 = phiT.reshape(24, E)
    dir16 = jnp.pad(jnp.stack(
        [dx.reshape(E), dy.reshape(E), dz.reshape(E)], axis=-1),
        ((0, 0), (0, 13)))

    vec = jnp.zeros((3, NP, F), jnp.float32)
    zeros_pf = jnp.zeros((NP, F), jnp.float32)

    for l in range(3):
        W_l = params['filter_W'][:, l * 3 * F:(l + 1) * 3 * F]
        b_l = params['filter_b'][l * 3 * F:(l + 1) * 3 * F]
        Waug = jnp.concatenate(
            [W_l, b_l[None], jnp.zeros((3, 3 * F), jnp.float32)], axis=0)
        h_i, filt = _pre(scalars, phiT2,
                         params['inter%d_W1' % l], params['inter%d_b1' % l][None],
                         params['inter%d_W2' % l], params['inter%d_b2' % l][None],
                         Waug)
        if l == 0:
            ds, d0, d1, d2 = _msg_nov(nbr_flat, filt, dir16, h_i,
                                      zeros_pf, zeros_pf, zeros_pf)
        else:
            ds, d0, d1, d2 = _msg_v(nbr_flat, filt, dir16, h_i,
                                    vec[0], vec[1], vec[2])
        dv = jnp.stack([d0, d1, d2])
        scalars, vec = _post(scalars, vec, ds, dv,
                             params['mu%d_W' % l],
                             params['intra%d_W1' % l], params['intra%d_b1' % l][None],
                             params['intra%d_W2' % l], params['intra%d_b2' % l][None])
    return scalars[:N][None]
